# baseline (speedup 1.0000x reference)
import functools

import jax
import jax.numpy as jnp
from jax import lax
from jax.experimental import pallas as pl
from jax.experimental.pallas import tpu as pltpu
from jax.experimental.pallas import tpu_sc as plsc

H = 2048
W = 2048
C = 3
N = 1048576

ROW = 8
NELEM = H * W * C
V = NELEM // ROW

NUM_CORES = 2
NUM_SUBCORES = 16
NUM_WORKERS = NUM_CORES * NUM_SUBCORES
LANES = 16

POINTS_PER_WORKER = N // NUM_WORKERS
CHUNK = 1024
NUM_CHUNKS = POINTS_PER_WORKER // CHUNK
GROUPS = CHUNK // LANES

BANDS = H // 8
BANDS_PER_WORKER = BANDS // NUM_WORKERS
BAND_IN = 16384
BAND_OUT = 3 * BAND_IN

_W_F = float(W)
_H_F = float(H)

_MESH = plsc.VectorSubcoreMesh(core_axis_name="c", subcore_axis_name="s")
_CPARAMS = pltpu.CompilerParams(needs_layout_passes=False,
                                use_tc_tiling_on_sc=False)


def _worker_id():
    return lax.axis_index("s") * NUM_CORES + lax.axis_index("c")


HALF_IN = BAND_IN // 2
HALF_ALL = C * HALF_IN
HALVES_PER_WORKER = 2 * BANDS_PER_WORKER


@functools.partial(
    pl.kernel,
    out_type=jax.ShapeDtypeStruct((NELEM,), jnp.float32),
    mesh=_MESH,
    compiler_params=_CPARAMS,
    scratch_types=[
        pltpu.VMEM((2, HALF_ALL), jnp.float32),
        pltpu.VMEM((2, HALF_ALL), jnp.float32),
        pltpu.SemaphoreType.DMA,
        pltpu.SemaphoreType.DMA,
        pltpu.SemaphoreType.DMA,
        pltpu.SemaphoreType.DMA,
    ],
)
def _interleave_sc(src_hbm, dst_hbm, in_v, out_v, isem0, isem1, osem0, osem1):
    wid = _worker_id()
    iota3 = 3 * lax.iota(jnp.int32, LANES)
    isem = (isem0, isem1)
    osem = (osem0, osem1)

    def in_copies(p, hh):
        ty = wid * BANDS_PER_WORKER + lax.shift_right_logical(hh, 1)
        h = jnp.bitwise_and(hh, 1)
        src0 = ty * BAND_IN + h * HALF_IN
        return [
            pltpu.make_async_copy(
                src_hbm.at[pl.ds(c * (H * W) + src0, HALF_IN)],
                in_v.at[p, pl.ds(c * HALF_IN, HALF_IN)],
                isem[p])
            for c in range(C)
        ]

    def out_copies(p, hh):
        ty = wid * BANDS_PER_WORKER + lax.shift_right_logical(hh, 1)
        h = jnp.bitwise_and(hh, 1)
        dst0 = ty * BAND_OUT + h * (C * HALF_IN // 8)
        return [
            pltpu.make_async_copy(
                out_v.at[p, pl.ds(yb * (C * 1024), C * 1024)],
                dst_hbm.at[pl.ds(dst0 + yb * (C * W), C * 1024)],
                osem[p])
            for yb in range(8)
        ]

    def shuffle(p):
        def sh_body(v2, carry2):
            tx = lax.shift_right_logical(v2, 6)
            rem = jnp.bitwise_and(v2, 63)
            yb = lax.shift_right_logical(rem, 3)
            xv = jnp.bitwise_and(rem, 7)
            base = yb * (C * 1024) + tx * (C * 128) + xv * (C * LANES)
            for c in range(C):
                src = in_v[p, pl.ds(c * HALF_IN + v2 * LANES, LANES)]
                plsc.store_scatter(out_v.at[p], [(base + c) + iota3], src)
            return carry2

        lax.fori_loop(0, HALF_IN // LANES, sh_body, 0, unroll=4)

    def stage(p, hh):
        for cp in in_copies(p, hh):
            cp.start()

    def drain_in(p, hh):
        for cp in in_copies(p, hh):
            cp.wait()

    def fire_out(p, hh):
        for cp in out_copies(p, hh):
            cp.start()

    def drain_out(p, hh):
        for cp in out_copies(p, hh):
            cp.wait()

    stage(0, 0)

    def pair_body(kk, carry):
        hh0 = 2 * kk
        hh1 = hh0 + 1
        stage(1, hh1)
        drain_in(0, hh0)

        @pl.when(kk > 0)
        def _():
            drain_out(0, hh0 - 2)

        shuffle(0)
        fire_out(0, hh0)

        @pl.when(hh1 + 1 < HALVES_PER_WORKER)
        def _():
            stage(0, hh1 + 1)

        drain_in(1, hh1)

        @pl.when(kk > 0)
        def _():
            drain_out(1, hh1 - 2)

        shuffle(1)
        fire_out(1, hh1)
        return carry

    lax.fori_loop(0, HALVES_PER_WORKER // 2, pair_body, 0)
    drain_out(0, HALVES_PER_WORKER - 2)
    drain_out(1, HALVES_PER_WORKER - 1)


def _bilerp_body(xs_hbm, data_hbm, out_hbm,
                 xs_v, it0, it1, ib0, ib1, ot_v, ob_v, wx_v, wy_v,
                 gt, gb, out_v, gsem0, gsem1, osem0, osem1):
    wid = _worker_id()
    base = wid * POINTS_PER_WORKER
    iota = lax.iota(jnp.int32, LANES)
    gsem = (gsem0, gsem1)
    osem = (osem0, osem1)

    def stage(p, k):
        off = base + k * CHUNK
        pltpu.sync_copy(xs_hbm.at[pl.ds(2 * off, 2 * CHUNK)], xs_v.at[p])

        def idx_body(j, carry2):
            sb = (j >> 3) * 256 + (j & 7) * LANES
            xf = xs_v[p, pl.ds(sb, LANES)]
            yf = xs_v[p, pl.ds(sb + 128, LANES)]
            sx = xf * _W_F
            sy = yf * _H_F
            x0 = sx.astype(jnp.int32)
            y0 = sy.astype(jnp.int32)
            wx = jnp.where(x0 == W - 1, 0.0, sx - x0.astype(jnp.float32))
            wy = sy - y0.astype(jnp.float32)
            y1 = jnp.minimum(y0 + 1, H - 1)
            e0 = (y0 * W + x0) * C
            e1 = (y1 * W + x0) * C
            r0 = lax.shift_right_logical(e0, 3)
            r1 = lax.shift_right_logical(e1, 3)
            sl = pl.ds(j * LANES, LANES)
            it0[p, sl] = r0
            it1[p, sl] = jnp.minimum(r0 + 1, V - 1)
            ib0[p, sl] = r1
            ib1[p, sl] = jnp.minimum(r1 + 1, V - 1)
            ot_v[p, sl] = jnp.bitwise_and(e0, 7)
            ob_v[p, sl] = jnp.bitwise_and(e1, 7)
            wx_v[p, sl] = wx
            wy_v[p, sl] = wy
            return carry2

        lax.fori_loop(0, GROUPS, idx_body, 0, unroll=2)

        pltpu.async_copy(data_hbm.at[it0.at[p]], gt.at[p, 0], gsem[p])
        pltpu.async_copy(data_hbm.at[it1.at[p]], gt.at[p, 1], gsem[p])
        pltpu.async_copy(data_hbm.at[ib0.at[p]], gb.at[p, 0], gsem[p])
        pltpu.async_copy(data_hbm.at[ib1.at[p]], gb.at[p, 1], gsem[p])

    def drain_gathers(p):
        pltpu.make_async_copy(data_hbm.at[it0.at[p]], gt.at[p, 0], gsem[p]).wait()
        pltpu.make_async_copy(data_hbm.at[it1.at[p]], gt.at[p, 1], gsem[p]).wait()
        pltpu.make_async_copy(data_hbm.at[ib0.at[p]], gb.at[p, 0], gsem[p]).wait()
        pltpu.make_async_copy(data_hbm.at[ib1.at[p]], gb.at[p, 1], gsem[p]).wait()

    def blend(p, k):
        gtp = gt.at[p]
        gbp = gb.at[p]

        def blend_body(j, carry2):
            sl = pl.ds(j * LANES, LANES)
            pos = j * LANES + iota
            wx = wx_v[p, sl]
            wy = wy_v[p, sl]
            ot = ot_v[p, sl]
            ob = ob_v[p, sl]
            w00 = (1.0 - wx) * (1.0 - wy)
            w01 = wx * (1.0 - wy)
            w10 = (1.0 - wx) * wy
            w11 = wx * wy
            gbase = (j >> 3) * 512 + (j & 7) * LANES
            for c in range(C):
                lt0 = ot + c
                lt1 = ot + (C + c)
                lb0 = ob + c
                lb1 = ob + (C + c)
                a = plsc.load_gather(
                    gtp, [lax.shift_right_logical(lt0, 3), pos,
                          jnp.bitwise_and(lt0, 7)])
                b = plsc.load_gather(
                    gtp, [lax.shift_right_logical(lt1, 3), pos,
                          jnp.bitwise_and(lt1, 7)])
                d = plsc.load_gather(
                    gbp, [lax.shift_right_logical(lb0, 3), pos,
                          jnp.bitwise_and(lb0, 7)])
                e = plsc.load_gather(
                    gbp, [lax.shift_right_logical(lb1, 3), pos,
                          jnp.bitwise_and(lb1, 7)])
                o = a * w00 + b * w01 + d * w10 + e * w11
                out_v[p, pl.ds(gbase + c * 128, LANES)] = o
            return carry2

        lax.fori_loop(0, GROUPS, blend_body, 0, unroll=4)
        off = base + k * CHUNK
        pltpu.async_copy(out_v.at[p], out_hbm.at[pl.ds(4 * off, 4 * CHUNK)],
                         osem[p])

    def drain_out(p, k):
        off = base + k * CHUNK
        pltpu.make_async_copy(out_v.at[p],
                              out_hbm.at[pl.ds(4 * off, 4 * CHUNK)],
                              osem[p]).wait()

    stage(0, 0)

    def pair_body(kk, carry):
        k0 = 2 * kk
        k1 = k0 + 1
        stage(1, k1)
        drain_gathers(0)

        @pl.when(kk > 0)
        def _():
            drain_out(0, k0 - 2)

        blend(0, k0)

        @pl.when(k1 + 1 < NUM_CHUNKS)
        def _():
            stage(0, k1 + 1)

        drain_gathers(1)

        @pl.when(kk > 0)
        def _():
            drain_out(1, k1 - 2)

        blend(1, k1)
        return carry

    lax.fori_loop(0, NUM_CHUNKS // 2, pair_body, 0)
    drain_out(0, NUM_CHUNKS - 2)
    drain_out(1, NUM_CHUNKS - 1)


@functools.partial(
    pl.kernel,
    out_type=jax.ShapeDtypeStruct((4 * N,), jnp.float32),
    mesh=_MESH,
    compiler_params=_CPARAMS,
    scratch_types=[
        pltpu.VMEM((2, 2 * CHUNK), jnp.float32),
        pltpu.VMEM((2, CHUNK), jnp.int32),
        pltpu.VMEM((2, CHUNK), jnp.int32),
        pltpu.VMEM((2, CHUNK), jnp.int32),
        pltpu.VMEM((2, CHUNK), jnp.int32),
        pltpu.VMEM((2, CHUNK), jnp.int32),
        pltpu.VMEM((2, CHUNK), jnp.int32),
        pltpu.VMEM((2, CHUNK), jnp.float32),
        pltpu.VMEM((2, CHUNK), jnp.float32),
        pltpu.VMEM((2, 2, CHUNK, ROW), jnp.float32),
        pltpu.VMEM((2, 2, CHUNK, ROW), jnp.float32),
        pltpu.VMEM((2, 4 * CHUNK), jnp.float32),
        pltpu.SemaphoreType.DMA,
        pltpu.SemaphoreType.DMA,
        pltpu.SemaphoreType.DMA,
        pltpu.SemaphoreType.DMA,
    ],
)
def _bilerp_sc(xs_hbm, data_hbm, out_hbm, *scratch):
    _bilerp_body(xs_hbm, data_hbm, out_hbm, *scratch)


def kernel(xs, data):
    xs_view = xs.T.reshape(2, N // 128, 128).transpose(1, 0, 2).reshape(-1)
    data_view = (data.transpose(2, 0, 1)
                 .reshape(C, H // 8, 8, W // 128, 128)
                 .transpose(0, 1, 3, 2, 4)
                 .reshape(-1))
    table = _interleave_sc(data_view).reshape(V, ROW)
    out1d = _bilerp_sc(xs_view, table)
    o = out1d.reshape(N // 128, 4, 128)[:, :C, :].transpose(0, 2, 1)
    return o.reshape(N, C)


# speedup vs baseline: 1.0098x; 1.0098x over previous
import functools

import jax
import jax.numpy as jnp
from jax import lax
from jax.experimental import pallas as pl
from jax.experimental.pallas import tpu as pltpu
from jax.experimental.pallas import tpu_sc as plsc

H = 2048
W = 2048
C = 3
N = 1048576

ROW = 8
NELEM = H * W * C
V = NELEM // ROW

NUM_CORES = 2
NUM_SUBCORES = 16
NUM_WORKERS = NUM_CORES * NUM_SUBCORES
LANES = 16

POINTS_PER_WORKER = N // NUM_WORKERS
CHUNK = 1024
NUM_CHUNKS = POINTS_PER_WORKER // CHUNK
GROUPS = CHUNK // LANES

BANDS = H // 8
BANDS_PER_WORKER = BANDS // NUM_WORKERS
BAND_IN = 16384
BAND_OUT = 3 * BAND_IN

_W_F = float(W)
_H_F = float(H)

_MESH = plsc.VectorSubcoreMesh(core_axis_name="c", subcore_axis_name="s")
_CPARAMS = pltpu.CompilerParams(needs_layout_passes=False,
                                use_tc_tiling_on_sc=False)


def _worker_id():
    return lax.axis_index("s") * NUM_CORES + lax.axis_index("c")


HALF_IN = BAND_IN // 2
HALF_ALL = C * HALF_IN
HALVES_PER_WORKER = 2 * BANDS_PER_WORKER


@functools.partial(
    pl.kernel,
    out_type=jax.ShapeDtypeStruct((NELEM,), jnp.float32),
    mesh=_MESH,
    compiler_params=_CPARAMS,
    scratch_types=[
        pltpu.VMEM((2, HALF_ALL), jnp.float32),
        pltpu.VMEM((2, HALF_ALL), jnp.float32),
        pltpu.SemaphoreType.DMA,
        pltpu.SemaphoreType.DMA,
        pltpu.SemaphoreType.DMA,
        pltpu.SemaphoreType.DMA,
    ],
)
def _interleave_sc(src_hbm, dst_hbm, in_v, out_v, isem0, isem1, osem0, osem1):
    wid = _worker_id()
    iota3 = 3 * lax.iota(jnp.int32, LANES)
    isem = (isem0, isem1)
    osem = (osem0, osem1)

    def in_copies(p, hh):
        ty = wid * BANDS_PER_WORKER + lax.shift_right_logical(hh, 1)
        h = jnp.bitwise_and(hh, 1)
        src0 = ty * BAND_IN + h * HALF_IN
        return [
            pltpu.make_async_copy(
                src_hbm.at[pl.ds(c * (H * W) + src0, HALF_IN)],
                in_v.at[p, pl.ds(c * HALF_IN, HALF_IN)],
                isem[p])
            for c in range(C)
        ]

    def out_copies(p, hh):
        ty = wid * BANDS_PER_WORKER + lax.shift_right_logical(hh, 1)
        h = jnp.bitwise_and(hh, 1)
        dst0 = ty * BAND_OUT + h * (C * HALF_IN // 8)
        return [
            pltpu.make_async_copy(
                out_v.at[p, pl.ds(yb * (C * 1024), C * 1024)],
                dst_hbm.at[pl.ds(dst0 + yb * (C * W), C * 1024)],
                osem[p])
            for yb in range(8)
        ]

    def shuffle(p):
        def sh_body(v2, carry2):
            tx = lax.shift_right_logical(v2, 6)
            rem = jnp.bitwise_and(v2, 63)
            yb = lax.shift_right_logical(rem, 3)
            xv = jnp.bitwise_and(rem, 7)
            base = yb * (C * 1024) + tx * (C * 128) + xv * (C * LANES)
            for c in range(C):
                src = in_v[p, pl.ds(c * HALF_IN + v2 * LANES, LANES)]
                plsc.store_scatter(out_v.at[p], [(base + c) + iota3], src)
            return carry2

        lax.fori_loop(0, HALF_IN // LANES, sh_body, 0, unroll=4)

    def stage(p, hh):
        for cp in in_copies(p, hh):
            cp.start()

    def drain_in(p, hh):
        for cp in in_copies(p, hh):
            cp.wait()

    def fire_out(p, hh):
        for cp in out_copies(p, hh):
            cp.start()

    def drain_out(p, hh):
        for cp in out_copies(p, hh):
            cp.wait()

    stage(0, 0)

    def pair_body(kk, carry):
        hh0 = 2 * kk
        hh1 = hh0 + 1
        stage(1, hh1)
        drain_in(0, hh0)

        @pl.when(kk > 0)
        def _():
            drain_out(0, hh0 - 2)

        shuffle(0)
        fire_out(0, hh0)

        @pl.when(hh1 + 1 < HALVES_PER_WORKER)
        def _():
            stage(0, hh1 + 1)

        drain_in(1, hh1)

        @pl.when(kk > 0)
        def _():
            drain_out(1, hh1 - 2)

        shuffle(1)
        fire_out(1, hh1)
        return carry

    lax.fori_loop(0, HALVES_PER_WORKER // 2, pair_body, 0)
    drain_out(0, HALVES_PER_WORKER - 2)
    drain_out(1, HALVES_PER_WORKER - 1)


def _bilerp_body(xs_hbm, data_hbm, out_hbm,
                 xs_v, it0, it1, ib0, ib1, ot_v, ob_v, wx_v, wy_v,
                 gt, gb, out_v, gsem0, gsem1, osem0, osem1, xsem0, xsem1):
    wid = _worker_id()
    base = wid * POINTS_PER_WORKER
    iota = lax.iota(jnp.int32, LANES)
    gsem = (gsem0, gsem1)
    osem = (osem0, osem1)
    xsem = (xsem0, xsem1)

    def xs_copy(p, k):
        off = base + k * CHUNK
        return pltpu.make_async_copy(xs_hbm.at[pl.ds(2 * off, 2 * CHUNK)],
                                     xs_v.at[p], xsem[p])

    def stage(p, k):
        xs_copy(p, k).wait()

        def idx_body(j, carry2):
            sb = (j >> 3) * 256 + (j & 7) * LANES
            xf = xs_v[p, pl.ds(sb, LANES)]
            yf = xs_v[p, pl.ds(sb + 128, LANES)]
            sx = xf * _W_F
            sy = yf * _H_F
            x0 = sx.astype(jnp.int32)
            y0 = sy.astype(jnp.int32)
            wx = jnp.where(x0 == W - 1, 0.0, sx - x0.astype(jnp.float32))
            wy = sy - y0.astype(jnp.float32)
            y1 = jnp.minimum(y0 + 1, H - 1)
            e0 = (y0 * W + x0) * C
            e1 = (y1 * W + x0) * C
            r0 = lax.shift_right_logical(e0, 3)
            r1 = lax.shift_right_logical(e1, 3)
            sl = pl.ds(j * LANES, LANES)
            it0[p, sl] = r0
            it1[p, sl] = jnp.minimum(r0 + 1, V - 1)
            ib0[p, sl] = r1
            ib1[p, sl] = jnp.minimum(r1 + 1, V - 1)
            ot_v[p, sl] = jnp.bitwise_and(e0, 7)
            ob_v[p, sl] = jnp.bitwise_and(e1, 7)
            wx_v[p, sl] = wx
            wy_v[p, sl] = wy
            return carry2

        lax.fori_loop(0, GROUPS, idx_body, 0, unroll=2)

        pltpu.async_copy(data_hbm.at[it0.at[p]], gt.at[p, 0], gsem[p])
        pltpu.async_copy(data_hbm.at[it1.at[p]], gt.at[p, 1], gsem[p])
        pltpu.async_copy(data_hbm.at[ib0.at[p]], gb.at[p, 0], gsem[p])
        pltpu.async_copy(data_hbm.at[ib1.at[p]], gb.at[p, 1], gsem[p])

    def drain_gathers(p):
        pltpu.make_async_copy(data_hbm.at[it0.at[p]], gt.at[p, 0], gsem[p]).wait()
        pltpu.make_async_copy(data_hbm.at[it1.at[p]], gt.at[p, 1], gsem[p]).wait()
        pltpu.make_async_copy(data_hbm.at[ib0.at[p]], gb.at[p, 0], gsem[p]).wait()
        pltpu.make_async_copy(data_hbm.at[ib1.at[p]], gb.at[p, 1], gsem[p]).wait()

    def blend(p, k):
        gtp = gt.at[p]
        gbp = gb.at[p]

        def blend_body(j, carry2):
            sl = pl.ds(j * LANES, LANES)
            pos = j * LANES + iota
            wx = wx_v[p, sl]
            wy = wy_v[p, sl]
            ot = ot_v[p, sl]
            ob = ob_v[p, sl]
            w00 = (1.0 - wx) * (1.0 - wy)
            w01 = wx * (1.0 - wy)
            w10 = (1.0 - wx) * wy
            w11 = wx * wy
            gbase = (j >> 3) * 512 + (j & 7) * LANES
            for c in range(C):
                lt0 = ot + c
                lt1 = ot + (C + c)
                lb0 = ob + c
                lb1 = ob + (C + c)
                a = plsc.load_gather(
                    gtp, [lax.shift_right_logical(lt0, 3), pos,
                          jnp.bitwise_and(lt0, 7)])
                b = plsc.load_gather(
                    gtp, [lax.shift_right_logical(lt1, 3), pos,
                          jnp.bitwise_and(lt1, 7)])
                d = plsc.load_gather(
                    gbp, [lax.shift_right_logical(lb0, 3), pos,
                          jnp.bitwise_and(lb0, 7)])
                e = plsc.load_gather(
                    gbp, [lax.shift_right_logical(lb1, 3), pos,
                          jnp.bitwise_and(lb1, 7)])
                o = a * w00 + b * w01 + d * w10 + e * w11
                out_v[p, pl.ds(gbase + c * 128, LANES)] = o
            return carry2

        lax.fori_loop(0, GROUPS, blend_body, 0, unroll=4)
        off = base + k * CHUNK
        pltpu.async_copy(out_v.at[p], out_hbm.at[pl.ds(4 * off, 4 * CHUNK)],
                         osem[p])

    def drain_out(p, k):
        off = base + k * CHUNK
        pltpu.make_async_copy(out_v.at[p],
                              out_hbm.at[pl.ds(4 * off, 4 * CHUNK)],
                              osem[p]).wait()

    def blend_with_drain(p, k):
        @pl.when(k >= 2)
        def _():
            drain_out(p, k - 2)

        blend(p, k)

    xs_copy(0, 0).start()

    def pair_body(kk, carry):
        k0 = 2 * kk
        k1 = k0 + 1
        stage(0, k0)
        xs_copy(1, k1).start()

        @pl.when(kk > 0)
        def _():
            drain_gathers(1)
            blend_with_drain(1, k0 - 1)

        stage(1, k1)

        @pl.when(k1 + 1 < NUM_CHUNKS)
        def _():
            xs_copy(0, k1 + 1).start()

        drain_gathers(0)
        blend_with_drain(0, k0)
        return carry

    lax.fori_loop(0, NUM_CHUNKS // 2, pair_body, 0)
    drain_gathers(1)
    blend_with_drain(1, NUM_CHUNKS - 1)
    drain_out(0, NUM_CHUNKS - 2)
    drain_out(1, NUM_CHUNKS - 1)


@functools.partial(
    pl.kernel,
    out_type=jax.ShapeDtypeStruct((4 * N,), jnp.float32),
    mesh=_MESH,
    compiler_params=_CPARAMS,
    scratch_types=[
        pltpu.VMEM((2, 2 * CHUNK), jnp.float32),
        pltpu.VMEM((2, CHUNK), jnp.int32),
        pltpu.VMEM((2, CHUNK), jnp.int32),
        pltpu.VMEM((2, CHUNK), jnp.int32),
        pltpu.VMEM((2, CHUNK), jnp.int32),
        pltpu.VMEM((2, CHUNK), jnp.int32),
        pltpu.VMEM((2, CHUNK), jnp.int32),
        pltpu.VMEM((2, CHUNK), jnp.float32),
        pltpu.VMEM((2, CHUNK), jnp.float32),
        pltpu.VMEM((2, 2, CHUNK, ROW), jnp.float32),
        pltpu.VMEM((2, 2, CHUNK, ROW), jnp.float32),
        pltpu.VMEM((2, 4 * CHUNK), jnp.float32),
        pltpu.SemaphoreType.DMA,
        pltpu.SemaphoreType.DMA,
        pltpu.SemaphoreType.DMA,
        pltpu.SemaphoreType.DMA,
        pltpu.SemaphoreType.DMA,
        pltpu.SemaphoreType.DMA,
    ],
)
def _bilerp_sc(xs_hbm, data_hbm, out_hbm, *scratch):
    _bilerp_body(xs_hbm, data_hbm, out_hbm, *scratch)


def kernel(xs, data):
    xs_view = xs.T.reshape(2, N // 128, 128).transpose(1, 0, 2).reshape(-1)
    data_view = (data.transpose(2, 0, 1)
                 .reshape(C, H // 8, 8, W // 128, 128)
                 .transpose(0, 1, 3, 2, 4)
                 .reshape(-1))
    table = _interleave_sc(data_view).reshape(V, ROW)
    out1d = _bilerp_sc(xs_view, table)
    o = out1d.reshape(N // 128, 4, 128)[:, :C, :].transpose(0, 2, 1)
    return o.reshape(N, C)
